# baseline (speedup 1.0000x reference)
import functools

import jax
import jax.numpy as jnp
from jax import lax
from jax.experimental import pallas as pl
from jax.experimental.pallas import tpu as pltpu
from jax.experimental.pallas import tpu_sc as plsc

_K = 100
_BLK = 512
_PM_LANES = 256
_M = 6144
_MW = _M + 16
_NW = 32
_LANES = 16


def _score_pm_kernel(q_ref, e_ref, s_ref, pm_ref, *, x_total):
    j = pl.program_id(0)
    s = jnp.dot(q_ref[...], e_ref[...], preferred_element_type=jnp.float32)
    col = j * _BLK + lax.broadcasted_iota(jnp.int32, s.shape, 1)
    s = jnp.where(col < x_total, s, -jnp.inf)
    s_ref[...] = s

    @pl.when(j == 0)
    def _():
        pm_ref[...] = jnp.full(pm_ref.shape, -jnp.inf, jnp.float32)

    m = jnp.max(s, axis=1, keepdims=True)
    lane = lax.broadcasted_iota(jnp.int32, pm_ref.shape, 1)
    pm_ref[...] = jnp.where(lane == j, m, pm_ref[...])


def _compact_kernel(scores_hbm, thr_hbm, val_hbm, idx_hbm,
                    row_v, thr_v, val_v, idx_v, *, rows, x_pad):
    n_vecs = x_pad // _LANES
    rows_per_w = rows // _NW
    wid = lax.axis_index("s") * 2 + lax.axis_index("c")

    def row_body(i, _):
        r = wid * rows_per_w + i
        pltpu.sync_copy(thr_hbm.at[pl.ds(r * _LANES, _LANES)], thr_v)
        pltpu.sync_copy(scores_hbm.at[pl.ds(r * x_pad, x_pad)], row_v)

        for s in range(_MW // _LANES):
            val_v[pl.ds(s * _LANES, _LANES)] = jnp.full(
                (_LANES,), -jnp.inf, jnp.float32)
            idx_v[pl.ds(s * _LANES, _LANES)] = jnp.zeros((_LANES,), jnp.int32)

        def vec_body(v, wp):
            x = row_v[pl.ds(v * _LANES, _LANES)]
            t = thr_v[...]
            msk = x >= t
            any_hit = jnp.max(msk.astype(jnp.int32))
            cols = v * _LANES + lax.iota(jnp.int32, _LANES)
            val_v[pl.ds(wp, _LANES)] = jnp.where(
                msk, x, jnp.full((_LANES,), -jnp.inf, jnp.float32))
            idx_v[pl.ds(wp, _LANES)] = cols
            return jnp.minimum(wp + _LANES * any_hit, _M)

        lax.fori_loop(0, n_vecs, vec_body, jnp.int32(0))
        pltpu.sync_copy(val_v, val_hbm.at[pl.ds(r * _MW, _MW)])
        pltpu.sync_copy(idx_v, idx_hbm.at[pl.ds(r * _MW, _MW)])
        return 0

    lax.fori_loop(0, rows_per_w, row_body, 0)


def kernel(query_embeddings, item_embeddings_t, item_ids, invalid_ids, k):
    b, d = query_embeddings.shape
    x = item_embeddings_t.shape[1]
    n_blocks = (x + _BLK - 1) // _BLK
    x_pad = n_blocks * _BLK
    kp = min(_K + invalid_ids.shape[1], item_ids.shape[1])

    e_pad = jnp.pad(item_embeddings_t, ((0, 0), (0, x_pad - x)))

    scores, pm = pl.pallas_call(
        functools.partial(_score_pm_kernel, x_total=x),
        grid=(n_blocks,),
        in_specs=[
            pl.BlockSpec((b, d), lambda j: (0, 0)),
            pl.BlockSpec((d, _BLK), lambda j: (0, j)),
        ],
        out_specs=[
            pl.BlockSpec((b, _BLK), lambda j: (0, j)),
            pl.BlockSpec((b, _PM_LANES), lambda j: (0, 0)),
        ],
        out_shape=[
            jax.ShapeDtypeStruct((b, x_pad), jnp.float32),
            jax.ShapeDtypeStruct((b, _PM_LANES), jnp.float32),
        ],
    )(query_embeddings, e_pad)

    top_vals, top_cols = lax.top_k(scores, kp)
    top_ids = jnp.take(item_ids[0], top_cols)

    valid = ~jnp.any(top_ids[:, :, None] == invalid_ids[:, None, :], axis=2)
    valid = jnp.logical_and(
        valid, jnp.cumsum(valid.astype(jnp.int32), axis=1) <= k
    )
    off = jnp.argsort(~valid, axis=1)[:, :_K]
    out_scores = jnp.take_along_axis(top_vals, off, axis=1)
    out_ids = jnp.take_along_axis(top_ids, off, axis=1)
    return (out_ids, out_scores)


# speedup vs baseline: 4.6346x; 4.6346x over previous
import functools

import jax
import jax.numpy as jnp
from jax import lax
from jax.experimental import pallas as pl

_K = 100
_BLK = 512
_G = 16
_MARGIN = 2


def _score_gmax_kernel(q_ref, e_ref, s_ref, gm_ref, *, x_total):
    j = pl.program_id(0)
    s = jnp.dot(q_ref[...], e_ref[...], preferred_element_type=jnp.float32)
    col = j * _BLK + lax.broadcasted_iota(jnp.int32, s.shape, 1)
    s = jnp.where(col < x_total, s, -jnp.inf)
    s_ref[...] = s

    b = s.shape[0]
    m = jnp.max(jnp.reshape(s, (b, _BLK // _G, _G)), axis=2)
    m4 = jnp.concatenate([m, m, m, m], axis=1)
    q = j % 4
    lane = lax.broadcasted_iota(jnp.int32, (b, 128), 1)
    quarter_mask = (lane >= q * 32) & (lane < (q + 1) * 32)

    @pl.when(q == 0)
    def _():
        gm_ref[...] = jnp.full((b, 128), -jnp.inf, jnp.float32)

    gm_ref[...] = jnp.where(quarter_mask, m4, gm_ref[...])


def kernel(query_embeddings, item_embeddings_t, item_ids, invalid_ids, k):
    b, d = query_embeddings.shape
    x = item_embeddings_t.shape[1]
    n_blocks = (x + _BLK - 1) // _BLK
    x_pad = n_blocks * _BLK
    n_groups = x_pad // _G
    kp = min(_K + invalid_ids.shape[1], item_ids.shape[1])
    k2 = min(_MARGIN * kp, n_groups)

    e_pad = jnp.pad(item_embeddings_t, ((0, 0), (0, x_pad - x)))

    scores, gmax = pl.pallas_call(
        functools.partial(_score_gmax_kernel, x_total=x),
        grid=(n_blocks,),
        in_specs=[
            pl.BlockSpec((b, d), lambda j: (0, 0)),
            pl.BlockSpec((d, _BLK), lambda j: (0, j)),
        ],
        out_specs=[
            pl.BlockSpec((b, _BLK), lambda j: (0, j)),
            pl.BlockSpec((b, 128), lambda j: (0, j // 4)),
        ],
        out_shape=[
            jax.ShapeDtypeStruct((b, x_pad), jnp.float32),
            jax.ShapeDtypeStruct((b, n_groups), jnp.float32),
        ],
    )(query_embeddings, e_pad)

    top_g = lax.top_k(gmax, k2)[1]
    s3 = jnp.reshape(scores, (b, n_groups, _G))
    cand_val = jnp.reshape(
        jnp.take_along_axis(s3, top_g[:, :, None], axis=1), (b, k2 * _G))
    cand_col = jnp.reshape(
        top_g[:, :, None] * _G + jnp.arange(_G, dtype=jnp.int32)[None, None, :],
        (b, k2 * _G))

    top_vals, top_pos = lax.top_k(cand_val, kp)
    top_cols = jnp.take_along_axis(cand_col, top_pos, axis=1)

    o1 = jnp.argsort(top_cols, axis=1, stable=True)
    v1 = jnp.take_along_axis(top_vals, o1, axis=1)
    c1 = jnp.take_along_axis(top_cols, o1, axis=1)
    o2 = jnp.argsort(-v1, axis=1, stable=True)
    top_vals = jnp.take_along_axis(v1, o2, axis=1)
    top_cols = jnp.take_along_axis(c1, o2, axis=1)

    top_ids = jnp.take(item_ids[0], top_cols)

    valid = ~jnp.any(top_ids[:, :, None] == invalid_ids[:, None, :], axis=2)
    valid = jnp.logical_and(
        valid, jnp.cumsum(valid.astype(jnp.int32), axis=1) <= k
    )
    off = jnp.argsort(~valid, axis=1)[:, :_K]
    out_scores = jnp.take_along_axis(top_vals, off, axis=1)
    out_ids = jnp.take_along_axis(top_ids, off, axis=1)
    return (out_ids, out_scores)


# speedup vs baseline: 5.1201x; 1.1047x over previous
import functools

import jax
import jax.numpy as jnp
from jax import lax
from jax.experimental import pallas as pl

_K = 100
_BLK = 512
_G = 16
_TIE_MARGIN = 18


def _score_gmax_kernel(q_ref, e_ref, s_ref, gm_ref, *, x_total):
    j = pl.program_id(0)
    s = jnp.dot(q_ref[...], e_ref[...], preferred_element_type=jnp.float32)
    col = j * _BLK + lax.broadcasted_iota(jnp.int32, s.shape, 1)
    s = jnp.where(col < x_total, s, -jnp.inf)
    s_ref[...] = s

    b = s.shape[0]
    m = jnp.max(jnp.reshape(s, (b, _BLK // _G, _G)), axis=2)
    m4 = jnp.concatenate([m, m, m, m], axis=1)
    q = j % 4
    lane = lax.broadcasted_iota(jnp.int32, (b, 128), 1)
    quarter_mask = (lane >= q * 32) & (lane < (q + 1) * 32)

    @pl.when(q == 0)
    def _():
        gm_ref[...] = jnp.full((b, 128), -jnp.inf, jnp.float32)

    gm_ref[...] = jnp.where(quarter_mask, m4, gm_ref[...])


def kernel(query_embeddings, item_embeddings_t, item_ids, invalid_ids, k):
    b, d = query_embeddings.shape
    x = item_embeddings_t.shape[1]
    n_blocks = (x + _BLK - 1) // _BLK
    x_pad = n_blocks * _BLK
    n_groups = x_pad // _G
    kp = min(_K + invalid_ids.shape[1], item_ids.shape[1])
    k2 = min(kp + _TIE_MARGIN, n_groups)

    e_pad = jnp.pad(item_embeddings_t, ((0, 0), (0, x_pad - x)))

    scores, gmax = pl.pallas_call(
        functools.partial(_score_gmax_kernel, x_total=x),
        grid=(n_blocks,),
        in_specs=[
            pl.BlockSpec((b, d), lambda j: (0, 0)),
            pl.BlockSpec((d, _BLK), lambda j: (0, j)),
        ],
        out_specs=[
            pl.BlockSpec((b, _BLK), lambda j: (0, j)),
            pl.BlockSpec((b, 128), lambda j: (0, j // 4)),
        ],
        out_shape=[
            jax.ShapeDtypeStruct((b, x_pad), jnp.float32),
            jax.ShapeDtypeStruct((b, n_groups), jnp.float32),
        ],
    )(query_embeddings, e_pad)

    top_g = lax.top_k(gmax, k2)[1]
    s3 = jnp.reshape(scores, (b, n_groups, _G))
    cand_val = jnp.reshape(
        jnp.take_along_axis(s3, top_g[:, :, None], axis=1), (b, k2 * _G))
    cand_col = jnp.reshape(
        top_g[:, :, None] * _G + jnp.arange(_G, dtype=jnp.int32)[None, None, :],
        (b, k2 * _G))

    top_vals, top_pos = lax.top_k(cand_val, kp)
    top_cols = jnp.take_along_axis(cand_col, top_pos, axis=1)

    o1 = jnp.argsort(top_cols, axis=1, stable=True)
    v1 = jnp.take_along_axis(top_vals, o1, axis=1)
    c1 = jnp.take_along_axis(top_cols, o1, axis=1)
    o2 = jnp.argsort(-v1, axis=1, stable=True)
    top_vals = jnp.take_along_axis(v1, o2, axis=1)
    top_cols = jnp.take_along_axis(c1, o2, axis=1)

    top_ids = jnp.take(item_ids[0], top_cols)

    valid = ~jnp.any(top_ids[:, :, None] == invalid_ids[:, None, :], axis=2)
    valid = jnp.logical_and(
        valid, jnp.cumsum(valid.astype(jnp.int32), axis=1) <= k
    )
    off = jnp.argsort(~valid, axis=1)[:, :_K]
    out_scores = jnp.take_along_axis(top_vals, off, axis=1)
    out_ids = jnp.take_along_axis(top_ids, off, axis=1)
    return (out_ids, out_scores)
